# Initial kernel scaffold; baseline (speedup 1.0000x reference)
import jax
import jax.numpy as jnp
from jax.experimental import pallas as pl


def kernel(adj_indices, adj_values, uEmbeds, iEmbeds, uHyperEmbeds, iHyperEmbeds):
    raise NotImplementedError("write your pallas kernel here")


# speedup vs baseline: 4.5498x; 4.5498x over previous
import functools

import jax
import jax.numpy as jnp
from jax import lax
from jax.experimental import pallas as pl
from jax.experimental.pallas import tpu as pltpu
from jax.experimental.pallas import tpu_sc as plsc

USER = 25000
ITEM = 25000
N = USER + ITEM
D = 64
HD = 32
HYP = 128
E = 800000

NC = 2
NS = 16
BLK = 128
STAGE = 16
TILE_BLOCKS = 400
STAGES = TILE_BLOCKS // STAGE
E_PAD = NS * TILE_BLOCKS * BLK
NBLK = E_PAD // BLK
STRIPE = 3128
LAST_STRIPE = N - (NS - 1) * STRIPE


def _leaky(x):
    return jnp.where(x >= 0, x, 0.5 * x)


def _spmm_body(cols_hbm, rows_hbm, vals_hbm, table_hbm, zeros_hbm, out_hbm,
               colsv, rowsv, valsv, msg, acc, gsem):
    c = lax.axis_index("c")
    s = lax.axis_index("s")

    def striped(fn):
        @pl.when(s < NS - 1)
        def _():
            fn(pl.ds(s * STRIPE, STRIPE))

        @pl.when(s == NS - 1)
        def _():
            fn(pl.ds((NS - 1) * STRIPE, LAST_STRIPE))

    striped(lambda sl: pltpu.sync_copy(zeros_hbm.at[sl], acc.at[sl]))
    plsc.subcore_barrier()

    def stage_body(st, carry):
        base = s * TILE_BLOCKS + st * STAGE
        pltpu.sync_copy(cols_hbm.at[pl.ds(base, STAGE)], colsv)
        pltpu.sync_copy(rows_hbm.at[pl.ds(base, STAGE)], rowsv)
        pltpu.sync_copy(vals_hbm.at[pl.ds(base, STAGE)], valsv)

        def block_body(j, carry2):
            pltpu.async_copy(table_hbm.at[c].at[colsv.at[j]], msg, gsem).wait()

            @plsc.parallel_loop(0, BLK, step=16)
            def _(e0):
                vv = valsv[j, pl.ds(e0, 16)]
                for k in range(16):
                    v = vv[k]
                    msg[e0 + k, pl.ds(0, 16)] = msg[e0 + k, pl.ds(0, 16)] * v
                    msg[e0 + k, pl.ds(16, 16)] = msg[e0 + k, pl.ds(16, 16)] * v

            pltpu.sync_copy(msg, acc.at[rowsv.at[j]], add=True)
            return carry2

        lax.fori_loop(0, STAGE, block_body, 0)
        return carry

    lax.fori_loop(0, STAGES, stage_body, 0)
    plsc.subcore_barrier()
    striped(lambda sl: pltpu.sync_copy(acc.at[sl], out_hbm.at[c].at[sl]))


_spmm = pl.kernel(
    _spmm_body,
    out_type=jax.ShapeDtypeStruct((NC, N, HD), jnp.float32),
    mesh=plsc.VectorSubcoreMesh(
        core_axis_name="c", subcore_axis_name="s",
        num_cores=NC, num_subcores=NS),
    compiler_params=pltpu.CompilerParams(use_tc_tiling_on_sc=False),
    scratch_types=[
        pltpu.VMEM((STAGE, BLK), jnp.int32),
        pltpu.VMEM((STAGE, BLK), jnp.int32),
        pltpu.VMEM((STAGE, BLK), jnp.float32),
        pltpu.VMEM((BLK, HD), jnp.float32),
        pltpu.VMEM_SHARED((N, HD), jnp.float32),
        pltpu.SemaphoreType.DMA,
    ],
)


RB = 1000


def _mm_body(x_ref, w_ref, o_ref):
    o_ref[...] = jnp.dot(x_ref[...], w_ref[...],
                         preferred_element_type=jnp.float32)


def _tc_matmul(x, w):
    rows = x.shape[0]
    return pl.pallas_call(
        _mm_body,
        grid=(rows // RB,),
        in_specs=[pl.BlockSpec((RB, D), lambda i: (i, 0)),
                  pl.BlockSpec((D, HYP), lambda i: (0, 0))],
        out_specs=pl.BlockSpec((RB, HYP), lambda i: (i, 0)),
        out_shape=jax.ShapeDtypeStruct((rows, HYP), jnp.float32),
    )(x, w)


def _hx_body(e_ref, h_ref, o_ref):
    i = pl.program_id(0)

    @pl.when(i == 0)
    def _():
        o_ref[...] = jnp.zeros_like(o_ref)

    o_ref[...] += lax.dot_general(
        e_ref[...], h_ref[...], (((0,), (0,)), ((), ())),
        preferred_element_type=jnp.float32)

    @pl.when(i == pl.num_programs(0) - 1)
    def _():
        o_ref[...] = _leaky(o_ref[...])


def _hyper_x(embs, hyper):
    rows = embs.shape[0]
    return pl.pallas_call(
        _hx_body,
        grid=(rows // RB,),
        in_specs=[pl.BlockSpec((RB, D), lambda i: (i, 0)),
                  pl.BlockSpec((RB, HYP), lambda i: (i, 0))],
        out_specs=pl.BlockSpec((D, HYP), lambda i: (0, 0)),
        out_shape=jax.ShapeDtypeStruct((D, HYP), jnp.float32),
    )(embs, hyper)


def _hn_body(h_ref, x_ref, s_ref, a_ref, new_ref, tot_ref):
    y = lax.dot_general(h_ref[...], x_ref[...], (((1,), (1,)), ((), ())),
                        preferred_element_type=jnp.float32)
    nv = _leaky(y) + s_ref[...]
    new_ref[...] = nv
    tot_ref[...] = a_ref[...] + nv


def _hyper_new(hyper, hx, spart, acc):
    rows = hyper.shape[0]
    return pl.pallas_call(
        _hn_body,
        grid=(rows // RB,),
        in_specs=[pl.BlockSpec((RB, HYP), lambda i: (i, 0)),
                  pl.BlockSpec((D, HYP), lambda i: (0, 0)),
                  pl.BlockSpec((RB, D), lambda i: (i, 0)),
                  pl.BlockSpec((RB, D), lambda i: (i, 0))],
        out_specs=[pl.BlockSpec((RB, D), lambda i: (i, 0)),
                   pl.BlockSpec((RB, D), lambda i: (i, 0))],
        out_shape=[jax.ShapeDtypeStruct((rows, D), jnp.float32),
                   jax.ShapeDtypeStruct((rows, D), jnp.float32)],
    )(hyper, hx, spart, acc)


def kernel(adj_indices, adj_values, uEmbeds, iEmbeds, uHyperEmbeds,
           iHyperEmbeds):
    rows = adj_indices[0].astype(jnp.int32)
    cols = adj_indices[1].astype(jnp.int32)
    vals = adj_values.astype(jnp.float32)

    pad = E_PAD - E
    spread = (jnp.arange(pad, dtype=jnp.int32) * 61) % N
    cols_p = jnp.concatenate([cols, spread]).reshape(NBLK, BLK)
    rows_p = jnp.concatenate([rows, spread]).reshape(NBLK, BLK)
    vals_p = jnp.concatenate(
        [vals, jnp.zeros((pad,), jnp.float32)]).reshape(NBLK, BLK)
    zeros = jnp.zeros((N, HD), jnp.float32)

    uu = _tc_matmul(uEmbeds, uHyperEmbeds)
    ii = _tc_matmul(iEmbeds, iHyperEmbeds)

    uPrev, iPrev = uEmbeds, iEmbeds
    uTot, iTot = uEmbeds, iEmbeds
    for _ in range(2):
        table = jnp.stack([
            jnp.concatenate([uPrev[:, :HD], iPrev[:, :HD]], axis=0),
            jnp.concatenate([uPrev[:, HD:], iPrev[:, HD:]], axis=0),
        ])
        sc_out = _spmm(cols_p, rows_p, vals_p, table, zeros)
        s_full = jnp.concatenate([sc_out[0], sc_out[1]], axis=1)
        uX = _hyper_x(uPrev, uu)
        iX = _hyper_x(iPrev, ii)
        uPrev, uTot = _hyper_new(uu, uX, s_full[:USER], uTot)
        iPrev, iTot = _hyper_new(ii, iX, s_full[USER:], iTot)
    return (uTot, iTot)


# speedup vs baseline: 6.1216x; 1.3455x over previous
import functools

import jax
import jax.numpy as jnp
from jax import lax
from jax.experimental import pallas as pl
from jax.experimental.pallas import tpu as pltpu
from jax.experimental.pallas import tpu_sc as plsc

USER = 25000
ITEM = 25000
N = USER + ITEM
D = 64
HD = 32
HYP = 128
E = 800000

NC = 2
NS = 16
BLK = 128
STAGE = 16
TILE_BLOCKS = 400
STAGES = TILE_BLOCKS // STAGE
E_PAD = NS * TILE_BLOCKS * BLK
NBLK = E_PAD // BLK
STRIPE = 3128
LAST_STRIPE = N - (NS - 1) * STRIPE


def _leaky(x):
    return jnp.where(x >= 0, x, 0.5 * x)


def _spmm_body(cols_hbm, rows_hbm, vals_hbm, table_hbm, zeros_hbm, out_hbm,
               colsv, rowsv, valsv, msg0, msg1, acc,
               gsem0, gsem1, ssem0, ssem1):
    c = lax.axis_index("c")
    s = lax.axis_index("s")

    def striped(fn):
        @pl.when(s < NS - 1)
        def _():
            fn(pl.ds(s * STRIPE, STRIPE))

        @pl.when(s == NS - 1)
        def _():
            fn(pl.ds((NS - 1) * STRIPE, LAST_STRIPE))

    striped(lambda sl: pltpu.sync_copy(zeros_hbm.at[sl], acc.at[sl]))
    plsc.subcore_barrier()

    def gather(j, buf, sem):
        pltpu.async_copy(table_hbm.at[c].at[colsv.at[j]], buf, sem)

    def gwait(buf, sem):
        pltpu.make_async_copy(table_hbm.at[0].at[colsv.at[0]], buf, sem).wait()

    def scale(buf, j):
        @plsc.parallel_loop(0, BLK, step=16)
        def _(e0):
            vv = valsv[j, pl.ds(e0, 16)]
            for k in range(16):
                v = vv[k]
                buf[e0 + k, pl.ds(0, 16)] = buf[e0 + k, pl.ds(0, 16)] * v
                buf[e0 + k, pl.ds(16, 16)] = buf[e0 + k, pl.ds(16, 16)] * v

    def scatter(j, buf, sem):
        pltpu.async_copy(buf, acc.at[rowsv.at[j]], sem, add=True)

    def swait(buf, sem):
        pltpu.make_async_copy(buf, acc.at[rowsv.at[0]], sem).wait()

    def stage_body(st, carry):
        base = s * TILE_BLOCKS + st * STAGE
        pltpu.sync_copy(cols_hbm.at[pl.ds(base, STAGE)], colsv)
        pltpu.sync_copy(rows_hbm.at[pl.ds(base, STAGE)], rowsv)
        pltpu.sync_copy(vals_hbm.at[pl.ds(base, STAGE)], valsv)

        gather(0, msg0, gsem0)

        def pair_body(jj, carry2):
            j0 = 2 * jj
            j1 = j0 + 1

            @pl.when(jj > 0)
            def _():
                swait(msg1, ssem1)

            gather(j1, msg1, gsem1)
            gwait(msg0, gsem0)
            scale(msg0, j0)
            scatter(j0, msg0, ssem0)

            @pl.when(jj < STAGE // 2 - 1)
            def _():
                swait(msg0, ssem0)
                gather(j0 + 2, msg0, gsem0)

            gwait(msg1, gsem1)
            scale(msg1, j1)
            scatter(j1, msg1, ssem1)
            return carry2

        lax.fori_loop(0, STAGE // 2, pair_body, 0)
        swait(msg0, ssem0)
        swait(msg1, ssem1)
        return carry

    lax.fori_loop(0, STAGES, stage_body, 0)
    plsc.subcore_barrier()
    striped(lambda sl: pltpu.sync_copy(acc.at[sl], out_hbm.at[c].at[sl]))


_spmm = pl.kernel(
    _spmm_body,
    out_type=jax.ShapeDtypeStruct((NC, N, HD), jnp.float32),
    mesh=plsc.VectorSubcoreMesh(
        core_axis_name="c", subcore_axis_name="s",
        num_cores=NC, num_subcores=NS),
    compiler_params=pltpu.CompilerParams(use_tc_tiling_on_sc=False),
    scratch_types=[
        pltpu.VMEM((STAGE, BLK), jnp.int32),
        pltpu.VMEM((STAGE, BLK), jnp.int32),
        pltpu.VMEM((STAGE, BLK), jnp.float32),
        pltpu.VMEM((BLK, HD), jnp.float32),
        pltpu.VMEM((BLK, HD), jnp.float32),
        pltpu.VMEM_SHARED((N, HD), jnp.float32),
        pltpu.SemaphoreType.DMA,
        pltpu.SemaphoreType.DMA,
        pltpu.SemaphoreType.DMA,
        pltpu.SemaphoreType.DMA,
    ],
)


RB = 1000


def _mm_body(x_ref, w_ref, o_ref):
    o_ref[...] = jnp.dot(x_ref[...], w_ref[...],
                         preferred_element_type=jnp.float32)


def _tc_matmul(x, w):
    rows = x.shape[0]
    return pl.pallas_call(
        _mm_body,
        grid=(rows // RB,),
        in_specs=[pl.BlockSpec((RB, D), lambda i: (i, 0)),
                  pl.BlockSpec((D, HYP), lambda i: (0, 0))],
        out_specs=pl.BlockSpec((RB, HYP), lambda i: (i, 0)),
        out_shape=jax.ShapeDtypeStruct((rows, HYP), jnp.float32),
    )(x, w)


def _hx_body(e_ref, h_ref, o_ref):
    i = pl.program_id(0)

    @pl.when(i == 0)
    def _():
        o_ref[...] = jnp.zeros_like(o_ref)

    o_ref[...] += lax.dot_general(
        e_ref[...], h_ref[...], (((0,), (0,)), ((), ())),
        preferred_element_type=jnp.float32)

    @pl.when(i == pl.num_programs(0) - 1)
    def _():
        o_ref[...] = _leaky(o_ref[...])


def _hyper_x(embs, hyper):
    rows = embs.shape[0]
    return pl.pallas_call(
        _hx_body,
        grid=(rows // RB,),
        in_specs=[pl.BlockSpec((RB, D), lambda i: (i, 0)),
                  pl.BlockSpec((RB, HYP), lambda i: (i, 0))],
        out_specs=pl.BlockSpec((D, HYP), lambda i: (0, 0)),
        out_shape=jax.ShapeDtypeStruct((D, HYP), jnp.float32),
    )(embs, hyper)


def _hn_body(h_ref, x_ref, s_ref, a_ref, new_ref, tot_ref):
    y = lax.dot_general(h_ref[...], x_ref[...], (((1,), (1,)), ((), ())),
                        preferred_element_type=jnp.float32)
    nv = _leaky(y) + s_ref[...]
    new_ref[...] = nv
    tot_ref[...] = a_ref[...] + nv


def _hyper_new(hyper, hx, spart, acc):
    rows = hyper.shape[0]
    return pl.pallas_call(
        _hn_body,
        grid=(rows // RB,),
        in_specs=[pl.BlockSpec((RB, HYP), lambda i: (i, 0)),
                  pl.BlockSpec((D, HYP), lambda i: (0, 0)),
                  pl.BlockSpec((RB, D), lambda i: (i, 0)),
                  pl.BlockSpec((RB, D), lambda i: (i, 0))],
        out_specs=[pl.BlockSpec((RB, D), lambda i: (i, 0)),
                   pl.BlockSpec((RB, D), lambda i: (i, 0))],
        out_shape=[jax.ShapeDtypeStruct((rows, D), jnp.float32),
                   jax.ShapeDtypeStruct((rows, D), jnp.float32)],
    )(hyper, hx, spart, acc)


def kernel(adj_indices, adj_values, uEmbeds, iEmbeds, uHyperEmbeds,
           iHyperEmbeds):
    rows = adj_indices[0].astype(jnp.int32)
    cols = adj_indices[1].astype(jnp.int32)
    vals = adj_values.astype(jnp.float32)

    pad = E_PAD - E
    spread = (jnp.arange(pad, dtype=jnp.int32) * 61) % N
    cols_p = jnp.concatenate([cols, spread]).reshape(NBLK, BLK)
    rows_p = jnp.concatenate([rows, spread]).reshape(NBLK, BLK)
    vals_p = jnp.concatenate(
        [vals, jnp.zeros((pad,), jnp.float32)]).reshape(NBLK, BLK)
    zeros = jnp.zeros((N, HD), jnp.float32)

    uu = _tc_matmul(uEmbeds, uHyperEmbeds)
    ii = _tc_matmul(iEmbeds, iHyperEmbeds)

    uPrev, iPrev = uEmbeds, iEmbeds
    uTot, iTot = uEmbeds, iEmbeds
    for _ in range(2):
        table = jnp.stack([
            jnp.concatenate([uPrev[:, :HD], iPrev[:, :HD]], axis=0),
            jnp.concatenate([uPrev[:, HD:], iPrev[:, HD:]], axis=0),
        ])
        sc_out = _spmm(cols_p, rows_p, vals_p, table, zeros)
        s_full = jnp.concatenate([sc_out[0], sc_out[1]], axis=1)
        uX = _hyper_x(uPrev, uu)
        iX = _hyper_x(iPrev, ii)
        uPrev, uTot = _hyper_new(uu, uX, s_full[:USER], uTot)
        iPrev, iTot = _hyper_new(ii, iX, s_full[USER:], iTot)
    return (uTot, iTot)
